# baseline (speedup 1.0000x reference)
import functools

import jax
import jax.numpy as jnp
from jax import lax
from jax.experimental import pallas as pl
from jax.experimental.pallas import tpu as pltpu
from jax.experimental.pallas import tpu_sc as plsc

WORD_DIM = 64
TAG_DIM = 32
OUT_DIM = WORD_DIM + TAG_DIM

R = 2
NBUF = 2
SLICE0 = 128
LANES = 16


def _build_kernel(B, L, num_cores, num_subcores):
  NW = num_cores * num_subcores
  rows_per_w = B // NW
  n_sb = rows_per_w // R
  n_body = n_sb // NBUF
  slice1 = L - SLICE0
  n_groups = (L + LANES - 1) // LANES

  mesh = plsc.VectorSubcoreMesh(core_axis_name="c", subcore_axis_name="s")

  @functools.partial(
      pl.kernel,
      mesh=mesh,
      out_type=jax.ShapeDtypeStruct((B, L, OUT_DIM), jnp.float32),
      compiler_params=pltpu.CompilerParams(
          use_tc_tiling_on_sc=False, needs_layout_passes=False),
      scratch_types=[
          pltpu.VMEM((NBUF, R, L), jnp.int32),
          pltpu.VMEM((NBUF, R, L), jnp.int32),
          pltpu.VMEM((NBUF, R, L, WORD_DIM), jnp.float32),
          pltpu.VMEM((NBUF, R, L, TAG_DIM), jnp.float32),
          pltpu.VMEM((64 * TAG_DIM,), jnp.float32),
          pltpu.SemaphoreType.DMA,
          pltpu.SemaphoreType.DMA,
          pltpu.SemaphoreType.DMA,
          pltpu.SemaphoreType.DMA,
      ],
  )
  def k(w_hbm, t_hbm, wt_hbm, tt_hbm, out_hbm,
        widx, tidx, wrows, trows, tagtab, g0, g1, o0, o1):
    c = lax.axis_index("c")
    s = lax.axis_index("s")
    wid = s * num_cores + c
    row_base = wid * rows_per_w
    gsem = (g0, g1)
    osem = (o0, o1)
    iota = lax.iota(jnp.int32, LANES)

    pltpu.sync_copy(tt_hbm, tagtab)

    def stage(sb, buf):
      b0 = row_base + sb * R
      pltpu.sync_copy(w_hbm.at[pl.ds(b0, R)], widx.at[buf])
      pltpu.sync_copy(t_hbm.at[pl.ds(b0, R)], tidx.at[buf])

    def fire(buf):
      copies = []
      for i in range(R):
        for (lo, ln) in ((0, SLICE0), (SLICE0, slice1)):
          copies.append(pltpu.async_copy(
              wt_hbm.at[widx.at[buf, i, pl.ds(lo, ln)]],
              wrows.at[buf, i, pl.ds(lo, ln)], gsem[buf]))
      return copies

    def tag_fill(buf):
      for i in range(R):
        def group(g, carry):
          start = jnp.minimum(g * LANES, L - LANES)
          tid = tidx[buf, i, pl.ds(start, LANES)]
          tok = start + iota
          for j in range(TAG_DIM):
            col = jnp.full((LANES,), j, jnp.int32)
            v = plsc.load_gather(tagtab, [tid * TAG_DIM + col])
            plsc.store_scatter(trows.at[buf, i], [tok, col], v)
          return carry
        lax.fori_loop(0, n_groups, group, 0)

    def write(sb, buf):
      b0 = row_base + sb * R
      return [
          pltpu.async_copy(
              wrows.at[buf],
              out_hbm.at[pl.ds(b0, R), slice(None), pl.ds(0, WORD_DIM)],
              osem[buf]),
          pltpu.async_copy(
              trows.at[buf],
              out_hbm.at[pl.ds(b0, R), slice(None), pl.ds(WORD_DIM, TAG_DIM)],
              osem[buf]),
      ]

    def body(i, carry):
      sb0 = i * NBUF
      sb1 = sb0 + 1
      stage(sb0, 0)
      c0 = fire(0)
      stage(sb1, 1)
      c1 = fire(1)
      tag_fill(0)
      tag_fill(1)
      for cp in c0:
        cp.wait()
      w0 = write(sb0, 0)
      for cp in c1:
        cp.wait()
      w1 = write(sb1, 1)
      for cp in w0 + w1:
        cp.wait()
      return carry

    lax.fori_loop(0, n_body, body, 0)

  return k


def kernel(words, tags, word_table, tag_table):
  B, L = words.shape
  info = plsc.get_sparse_core_info()
  k = _build_kernel(B, L, info.num_cores, info.num_subcores)
  return k(words, tags, word_table, tag_table.reshape(-1))


# speedup vs baseline: 1.6961x; 1.6961x over previous
import functools

import jax
import jax.numpy as jnp
from jax import lax
from jax.experimental import pallas as pl
from jax.experimental.pallas import tpu as pltpu
from jax.experimental.pallas import tpu_sc as plsc

WORD_DIM = 64
TAG_DIM = 32
OUT_DIM = WORD_DIM + TAG_DIM

R = 2
NBUF = 2
SLICE0 = 128
LANES = 16


def _build_kernel(B, L, num_cores, num_subcores):
  NW = num_cores * num_subcores
  rows_per_w = B // NW
  n_sb = rows_per_w // R
  n_body = n_sb // NBUF
  slice1 = L - SLICE0
  n_groups = (L + LANES - 1) // LANES

  mesh = plsc.VectorSubcoreMesh(core_axis_name="c", subcore_axis_name="s")

  @functools.partial(
      pl.kernel,
      mesh=mesh,
      out_type=jax.ShapeDtypeStruct((B, L, OUT_DIM), jnp.float32),
      compiler_params=pltpu.CompilerParams(
          use_tc_tiling_on_sc=False, needs_layout_passes=False),
      scratch_types=[
          pltpu.VMEM((NBUF, R, L), jnp.int32),
          pltpu.VMEM((NBUF, R, L), jnp.int32),
          pltpu.VMEM((NBUF, R, L, WORD_DIM), jnp.float32),
          pltpu.VMEM((NBUF, R, L, TAG_DIM), jnp.float32),
          pltpu.VMEM((64 * TAG_DIM,), jnp.float32),
          pltpu.SemaphoreType.DMA,
          pltpu.SemaphoreType.DMA,
          pltpu.SemaphoreType.DMA,
          pltpu.SemaphoreType.DMA,
      ],
  )
  def k(w_hbm, t_hbm, wt_hbm, tt_hbm, out_hbm,
        widx, tidx, wrows, trows, tagtab, g0, g1, o0, o1):
    c = lax.axis_index("c")
    s = lax.axis_index("s")
    wid = s * num_cores + c
    row_base = wid * rows_per_w
    gsem = (g0, g1)
    osem = (o0, o1)
    iota = lax.iota(jnp.int32, LANES)

    pltpu.sync_copy(tt_hbm, tagtab)

    def stage(sb, buf):
      b0 = row_base + sb * R
      pltpu.sync_copy(w_hbm.at[pl.ds(b0, R)], widx.at[buf])
      pltpu.sync_copy(t_hbm.at[pl.ds(b0, R)], tidx.at[buf])

    def fire(buf):
      copies = []
      for i in range(R):
        for (lo, ln) in ((0, SLICE0), (SLICE0, slice1)):
          copies.append(pltpu.async_copy(
              wt_hbm.at[widx.at[buf, i, pl.ds(lo, ln)]],
              wrows.at[buf, i, pl.ds(lo, ln)], gsem[buf]))
      return copies

    def tag_fill(buf):
      for i in range(R):
        def group(g, carry):
          start = jnp.minimum(g * LANES, L - LANES)
          tid16 = tidx[buf, i, pl.ds(start, LANES)]
          for lane in range(LANES):
            t = start + lane
            base = tid16[lane] * TAG_DIM
            trows[buf, i, t, pl.ds(0, LANES)] = tagtab[pl.ds(base, LANES)]
            trows[buf, i, t, pl.ds(LANES, LANES)] = (
                tagtab[pl.ds(base + LANES, LANES)])
          return carry
        lax.fori_loop(0, n_groups, group, 0)

    def write(sb, buf):
      b0 = row_base + sb * R
      return [
          pltpu.async_copy(
              wrows.at[buf],
              out_hbm.at[pl.ds(b0, R), slice(None), pl.ds(0, WORD_DIM)],
              osem[buf]),
          pltpu.async_copy(
              trows.at[buf],
              out_hbm.at[pl.ds(b0, R), slice(None), pl.ds(WORD_DIM, TAG_DIM)],
              osem[buf]),
      ]

    def body(i, carry):
      sb0 = i * NBUF
      sb1 = sb0 + 1
      stage(sb0, 0)
      c0 = fire(0)
      stage(sb1, 1)
      c1 = fire(1)
      tag_fill(0)
      tag_fill(1)
      for cp in c0:
        cp.wait()
      w0 = write(sb0, 0)
      for cp in c1:
        cp.wait()
      w1 = write(sb1, 1)
      for cp in w0 + w1:
        cp.wait()
      return carry

    lax.fori_loop(0, n_body, body, 0)

  return k


def kernel(words, tags, word_table, tag_table):
  B, L = words.shape
  info = plsc.get_sparse_core_info()
  k = _build_kernel(B, L, info.num_cores, info.num_subcores)
  return k(words, tags, word_table, tag_table.reshape(-1))


# speedup vs baseline: 1.7761x; 1.0472x over previous
import functools

import jax
import jax.numpy as jnp
from jax import lax
from jax.experimental import pallas as pl
from jax.experimental.pallas import tpu as pltpu
from jax.experimental.pallas import tpu_sc as plsc

WORD_DIM = 64
TAG_DIM = 32
OUT_DIM = WORD_DIM + TAG_DIM

R = 2
NBUF = 2
SLICE0 = 128
LANES = 16


def _build_kernel(B, L, num_cores, num_subcores):
  NW = num_cores * num_subcores
  rows_per_w = B // NW
  n_sb = rows_per_w // R
  n_body = n_sb // NBUF
  slice1 = L - SLICE0
  n_groups = (L + LANES - 1) // LANES

  mesh = plsc.VectorSubcoreMesh(core_axis_name="c", subcore_axis_name="s")

  @functools.partial(
      pl.kernel,
      mesh=mesh,
      out_type=jax.ShapeDtypeStruct((B, L, OUT_DIM), jnp.float32),
      compiler_params=pltpu.CompilerParams(
          use_tc_tiling_on_sc=False, needs_layout_passes=False),
      scratch_types=[
          pltpu.VMEM((NBUF * R, L), jnp.int32),
          pltpu.VMEM((NBUF * R, L), jnp.int32),
          pltpu.VMEM((NBUF, R, L, WORD_DIM), jnp.float32),
          pltpu.VMEM((NBUF, R, L, TAG_DIM), jnp.float32),
          pltpu.VMEM((64 * TAG_DIM,), jnp.float32),
          pltpu.SemaphoreType.DMA,
          pltpu.SemaphoreType.DMA,
          pltpu.SemaphoreType.DMA,
          pltpu.SemaphoreType.DMA,
      ],
  )
  def k(w_hbm, t_hbm, wt_hbm, tt_hbm, out_hbm,
        widx, tidx, wrows, trows, tagtab, g0, g1, o0, o1):
    c = lax.axis_index("c")
    s = lax.axis_index("s")
    wid = s * num_cores + c
    row_base = wid * rows_per_w
    gsem = (g0, g1)
    osem = (o0, o1)
    iota = lax.iota(jnp.int32, LANES)

    pltpu.sync_copy(tt_hbm, tagtab)

    def stage(body_i):
      b0 = row_base + body_i * NBUF * R
      cw = pltpu.async_copy(w_hbm.at[pl.ds(b0, NBUF * R)], widx, g0)
      ct = pltpu.async_copy(t_hbm.at[pl.ds(b0, NBUF * R)], tidx, g0)
      cw.wait()
      ct.wait()

    def fire(buf):
      copies = []
      for i in range(R):
        for (lo, ln) in ((0, SLICE0), (SLICE0, slice1)):
          copies.append(pltpu.async_copy(
              wt_hbm.at[widx.at[buf * R + i, pl.ds(lo, ln)]],
              wrows.at[buf, i, pl.ds(lo, ln)], gsem[buf]))
      return copies

    def tag_fill(buf):
      for i in range(R):
        def group(g, carry):
          start = jnp.minimum(g * LANES, L - LANES)
          tid16 = tidx[buf * R + i, pl.ds(start, LANES)]
          for lane in range(LANES):
            t = start + lane
            base = tid16[lane] * TAG_DIM
            trows[buf, i, t, pl.ds(0, LANES)] = tagtab[pl.ds(base, LANES)]
            trows[buf, i, t, pl.ds(LANES, LANES)] = (
                tagtab[pl.ds(base + LANES, LANES)])
          return carry
        lax.fori_loop(0, n_groups, group, 0)

    def write(sb, buf):
      b0 = row_base + sb * R
      return [
          pltpu.async_copy(
              wrows.at[buf],
              out_hbm.at[pl.ds(b0, R), slice(None), pl.ds(0, WORD_DIM)],
              osem[buf]),
          pltpu.async_copy(
              trows.at[buf],
              out_hbm.at[pl.ds(b0, R), slice(None), pl.ds(WORD_DIM, TAG_DIM)],
              osem[buf]),
      ]

    def body(i, carry):
      sb0 = i * NBUF
      sb1 = sb0 + 1
      stage(i)
      c0 = fire(0)
      c1 = fire(1)
      tag_fill(0)
      tag_fill(1)
      for cp in c0:
        cp.wait()
      w0 = write(sb0, 0)
      for cp in c1:
        cp.wait()
      w1 = write(sb1, 1)
      for cp in w0 + w1:
        cp.wait()
      return carry

    lax.fori_loop(0, n_body, body, 0)

  return k


def kernel(words, tags, word_table, tag_table):
  B, L = words.shape
  info = plsc.get_sparse_core_info()
  k = _build_kernel(B, L, info.num_cores, info.num_subcores)
  return k(words, tags, word_table, tag_table.reshape(-1))
